# baseline (speedup 1.0000x reference)
import functools

import jax
import jax.numpy as jnp
from jax import lax
from jax.experimental import pallas as pl
from jax.experimental.pallas import tpu as pltpu
from jax.experimental.pallas import tpu_sc as plsc

NC = 2
NS = 16
NW = NC * NS
LANES = 16
NPAD = 10240

_mesh = lambda: plsc.VectorSubcoreMesh(core_axis_name="c", subcore_axis_name="s")


def _sc_degree(dst, n_nodes):
    del n_nodes
    nwin = dst.shape[1]
    k = dst.shape[2]
    rpt = NPAD // NS
    nb = 5

    @functools.partial(
        pl.kernel,
        mesh=_mesh(),
        out_type=jax.ShapeDtypeStruct((NC, NPAD, LANES), jnp.float32),
        scratch_types=[
            pltpu.VMEM_SHARED((NPAD, LANES), jnp.float32),
            pltpu.VMEM((rpt, LANES), jnp.float32),
            pltpu.VMEM((k, LANES), jnp.float32),
            pltpu.VMEM((nwin, k), jnp.int32),
        ] + [pltpu.SemaphoreType.DMA] * nb,
    )
    def deg_kernel(dst_hbm, out_hbm, acc, zbuf, ones_v, idxall, *ssem):
        cid = lax.axis_index("c")
        sid = lax.axis_index("s")
        wid = cid * NS + sid

        @pl.loop(0, rpt)
        def _(r):
            zbuf[r, :] = jnp.zeros((LANES,), jnp.float32)

        @pl.loop(0, k)
        def _(r):
            ones_v[r, :] = jnp.ones((LANES,), jnp.float32)

        pltpu.sync_copy(dst_hbm.at[wid], idxall)
        pltpu.sync_copy(zbuf, acc.at[pl.ds(sid * rpt, rpt)])
        plsc.subcore_barrier()

        def scat_start(b, w):
            pltpu.async_copy(ones_v, acc.at[idxall.at[w]], ssem[b],
                             add=True)

        def scat_wait(b, w):
            pltpu.make_async_copy(ones_v, acc.at[idxall.at[w]],
                                  ssem[b]).wait()

        for b in range(nb):
            scat_start(b, b)

        @pl.loop(0, nwin // nb - 1)
        def _(g):
            w0 = g * nb
            for b in range(nb):
                scat_wait(b, w0 + b)
                scat_start(b, w0 + nb + b)

        wl = nwin - nb
        for b in range(nb):
            scat_wait(b, wl + b)

        plsc.subcore_barrier()
        pltpu.sync_copy(
            acc.at[pl.ds(sid * rpt, rpt)],
            out_hbm.at[cid].at[pl.ds(sid * rpt, rpt)],
        )

    return deg_kernel(dst)


def _sc_edge_pass(y, eidx):
    n_nodes, d = y.shape
    del n_nodes
    nwin = eidx.shape[1]
    k = eidx.shape[3]
    rpt = NPAD // NS
    nb = 5
    nph = 10
    nphase = nwin // nph

    @functools.partial(
        pl.kernel,
        mesh=_mesh(),
        out_type=jax.ShapeDtypeStruct((NC, NPAD, d), jnp.float32),
        scratch_types=[
            pltpu.VMEM_SHARED((NPAD, d), jnp.float32),
            pltpu.VMEM((2, nphase, 2, k), jnp.int32),
            pltpu.VMEM((nb, k, d), jnp.float32),
        ] + [pltpu.SemaphoreType.DMA] * (2 * nb + 2),
    )
    def edge_kernel(y_hbm, eidx_hbm, out_hbm, acc, idxs, rows, *sems):
        gsem, ssem, isem = sems[:nb], sems[nb:2 * nb], sems[2 * nb:]
        cid = lax.axis_index("c")
        sid = lax.axis_index("s")
        wid = cid * NS + sid

        def idx_start(pb, p):
            pltpu.async_copy(
                eidx_hbm.at[wid].at[pl.ds(p * nphase, nphase)],
                idxs.at[pb], isem[pb])

        def idx_wait(pb, p):
            pltpu.make_async_copy(
                eidx_hbm.at[wid].at[pl.ds(p * nphase, nphase)],
                idxs.at[pb], isem[pb]).wait()

        idx_start(0, 0)

        @pl.loop(0, k)
        def _(r):
            @pl.loop(0, d // LANES)
            def _(cc):
                rows[0, r, pl.ds(cc * LANES, LANES)] = jnp.zeros(
                    (LANES,), jnp.float32)

        @pl.loop(0, rpt // k)
        def _(z):
            pltpu.sync_copy(rows.at[0], acc.at[pl.ds(sid * rpt + z * k, k)])

        plsc.subcore_barrier()

        def gath_start(pb, b, w):
            pltpu.async_copy(y_hbm.at[idxs.at[pb].at[w].at[0]], rows.at[b],
                             gsem[b])

        def gath_wait(pb, b, w):
            pltpu.make_async_copy(y_hbm.at[idxs.at[pb].at[w].at[0]],
                                  rows.at[b], gsem[b]).wait()

        def scat_start(pb, b, w):
            pltpu.async_copy(rows.at[b], acc.at[idxs.at[pb].at[w].at[1]],
                             ssem[b], add=True)

        def scat_wait(pb, b, w):
            pltpu.make_async_copy(rows.at[b], acc.at[idxs.at[pb].at[w].at[1]],
                                  ssem[b]).wait()

        for p in range(nph):
            pb = p % 2
            idx_wait(pb, p)
            if p + 1 < nph:
                idx_start(1 - pb, p + 1)

            for b in range(nb):
                gath_start(pb, b, b)

            @pl.loop(0, nphase // nb - 1)
            def _(g):
                w0 = g * nb
                for b in range(nb):
                    gath_wait(pb, b, w0 + b)
                    scat_start(pb, b, w0 + b)
                for b in range(nb):
                    scat_wait(pb, b, w0 + b)
                    gath_start(pb, b, w0 + nb + b)

            wl = nphase - nb
            for b in range(nb):
                gath_wait(pb, b, wl + b)
                scat_start(pb, b, wl + b)
            for b in range(nb):
                scat_wait(pb, b, wl + b)

        plsc.subcore_barrier()
        pltpu.sync_copy(
            acc.at[pl.ds(sid * rpt, rpt)],
            out_hbm.at[cid].at[pl.ds(sid * rpt, rpt)],
        )

    return edge_kernel(y, eidx)


_R = 1000


def _tc_matmul(x, w):
    n, d = x.shape

    def body(x_ref, w_ref, u_ref):
        u_ref[...] = jnp.dot(x_ref[...], w_ref[...],
                             preferred_element_type=jnp.float32)

    return pl.pallas_call(
        body,
        grid=(n // _R,),
        in_specs=[
            pl.BlockSpec((_R, d), lambda i: (i, 0)),
            pl.BlockSpec((d, d), lambda i: (0, 0)),
        ],
        out_specs=pl.BlockSpec((_R, d), lambda i: (i, 0)),
        out_shape=jax.ShapeDtypeStruct((n, d), jnp.float32),
    )(x, w)


def _tc_scale(u, degc):
    n, d = u.shape

    def body(u_ref, d0_ref, d1_ref, y_ref, dinv_ref):
        deg = d0_ref[0, :, 0:1] + d1_ref[0, :, 0:1] + 2.0
        dinv = lax.rsqrt(deg)
        dinv_ref[...] = dinv
        y_ref[...] = u_ref[...] * dinv

    return pl.pallas_call(
        body,
        grid=(n // _R,),
        in_specs=[
            pl.BlockSpec((_R, d), lambda i: (i, 0)),
            pl.BlockSpec((1, _R, LANES), lambda i: (0, i, 0)),
            pl.BlockSpec((1, _R, LANES), lambda i: (1, i, 0)),
        ],
        out_specs=[
            pl.BlockSpec((_R, d), lambda i: (i, 0)),
            pl.BlockSpec((_R, 1), lambda i: (i, 0)),
        ],
        out_shape=[
            jax.ShapeDtypeStruct((n, d), jnp.float32),
            jax.ShapeDtypeStruct((n, 1), jnp.float32),
        ],
    )(u, degc, degc)


def _tc_tmp(y, res, b, dinv):
    n, d = y.shape

    def body(y_ref, res_ref, b_ref, dinv_ref, t_ref):
        t_ref[...] = 2.0 * dinv_ref[...] * y_ref[...] \
            + b_ref[...][None, :] + res_ref[...]

    return pl.pallas_call(
        body,
        grid=(n // _R,),
        in_specs=[
            pl.BlockSpec((_R, d), lambda i: (i, 0)),
            pl.BlockSpec((_R, d), lambda i: (i, 0)),
            pl.BlockSpec((d,), lambda i: (0,)),
            pl.BlockSpec((_R, 1), lambda i: (i, 0)),
        ],
        out_specs=pl.BlockSpec((_R, d), lambda i: (i, 0)),
        out_shape=jax.ShapeDtypeStruct((n, d), jnp.float32),
    )(y, res, b, dinv)


def _tc_combine_matmul(part, tmp, w, dinv):
    n, d = tmp.shape

    def body(p0_ref, p1_ref, t_ref, w_ref, dinv_ref, h_ref, y2_ref):
        dinv = dinv_ref[...]
        h = dinv * (p0_ref[0] + p1_ref[0]) + t_ref[...]
        h_ref[...] = h
        y2_ref[...] = jnp.dot(h, w_ref[...],
                              preferred_element_type=jnp.float32) * dinv

    return pl.pallas_call(
        body,
        grid=(n // _R,),
        in_specs=[
            pl.BlockSpec((1, _R, d), lambda i: (0, i, 0)),
            pl.BlockSpec((1, _R, d), lambda i: (1, i, 0)),
            pl.BlockSpec((_R, d), lambda i: (i, 0)),
            pl.BlockSpec((d, d), lambda i: (0, 0)),
            pl.BlockSpec((_R, 1), lambda i: (i, 0)),
        ],
        out_specs=[
            pl.BlockSpec((_R, d), lambda i: (i, 0)),
            pl.BlockSpec((_R, d), lambda i: (i, 0)),
        ],
        out_shape=[
            jax.ShapeDtypeStruct((n, d), jnp.float32),
            jax.ShapeDtypeStruct((n, d), jnp.float32),
        ],
    )(part, part, tmp, w, dinv)


def _tc_combine(part, tmp, dinv):
    n, d = tmp.shape

    def body(p0_ref, p1_ref, t_ref, dinv_ref, o_ref):
        o_ref[...] = dinv_ref[...] * (p0_ref[0] + p1_ref[0]) + t_ref[...]

    return pl.pallas_call(
        body,
        grid=(n // _R,),
        in_specs=[
            pl.BlockSpec((1, _R, d), lambda i: (0, i, 0)),
            pl.BlockSpec((1, _R, d), lambda i: (1, i, 0)),
            pl.BlockSpec((_R, d), lambda i: (i, 0)),
            pl.BlockSpec((_R, 1), lambda i: (i, 0)),
        ],
        out_specs=pl.BlockSpec((_R, d), lambda i: (i, 0)),
        out_shape=jax.ShapeDtypeStruct((n, d), jnp.float32),
    )(part, part, tmp, dinv)


def kernel(x, edge_index, W1, b1, W2, b2):
    e = edge_index.shape[1]
    k = 40
    nwin = e // (NW * k)
    eidx = jnp.stack(
        [edge_index[0].reshape(NW, nwin, k),
         edge_index[1].reshape(NW, nwin, k)], axis=2)

    degc = _sc_degree(edge_index[1].reshape(NW, nwin, k), x.shape[0])
    u1 = _tc_matmul(x, W1)
    y1, dinv = _tc_scale(u1, degc)
    p1 = _sc_edge_pass(y1, eidx)
    t2 = _tc_tmp(y1, x, b1, dinv)
    h, y2 = _tc_combine_matmul(p1, t2, W2, dinv)
    p2 = _sc_edge_pass(y2, eidx)
    t3 = _tc_tmp(y2, h, b2, dinv)
    out = _tc_combine(p2, t3, dinv)
    return out


# speedup vs baseline: 1.0226x; 1.0226x over previous
import functools

import jax
import jax.numpy as jnp
from jax import lax
from jax.experimental import pallas as pl
from jax.experimental.pallas import tpu as pltpu
from jax.experimental.pallas import tpu_sc as plsc

NC = 2
NS = 16
NW = NC * NS
LANES = 16
NPAD = 10240

_mesh = lambda: plsc.VectorSubcoreMesh(core_axis_name="c", subcore_axis_name="s")


def _sc_degree(dst, n_nodes):
    del n_nodes
    nwin = dst.shape[1]
    k = dst.shape[2]
    rpt = NPAD // NS
    nb = 5

    @functools.partial(
        pl.kernel,
        mesh=_mesh(),
        out_type=jax.ShapeDtypeStruct((NC, NPAD, LANES), jnp.float32),
        scratch_types=[
            pltpu.VMEM_SHARED((NPAD, LANES), jnp.float32),
            pltpu.VMEM((rpt, LANES), jnp.float32),
            pltpu.VMEM((k, LANES), jnp.float32),
            pltpu.VMEM((nwin, k), jnp.int32),
        ] + [pltpu.SemaphoreType.DMA] * nb,
    )
    def deg_kernel(dst_hbm, out_hbm, acc, zbuf, ones_v, idxall, *ssem):
        cid = lax.axis_index("c")
        sid = lax.axis_index("s")
        wid = cid * NS + sid

        @pl.loop(0, rpt)
        def _(r):
            zbuf[r, :] = jnp.zeros((LANES,), jnp.float32)

        @pl.loop(0, k)
        def _(r):
            ones_v[r, :] = jnp.ones((LANES,), jnp.float32)

        pltpu.sync_copy(dst_hbm.at[wid], idxall)
        pltpu.sync_copy(zbuf, acc.at[pl.ds(sid * rpt, rpt)])
        plsc.subcore_barrier()

        def scat_start(b, w):
            pltpu.async_copy(ones_v, acc.at[idxall.at[w]], ssem[b],
                             add=True)

        def scat_wait(b, w):
            pltpu.make_async_copy(ones_v, acc.at[idxall.at[w]],
                                  ssem[b]).wait()

        for b in range(nb):
            scat_start(b, b)

        @pl.loop(0, nwin // nb - 1)
        def _(g):
            w0 = g * nb
            for b in range(nb):
                scat_wait(b, w0 + b)
                scat_start(b, w0 + nb + b)

        wl = nwin - nb
        for b in range(nb):
            scat_wait(b, wl + b)

        plsc.subcore_barrier()
        pltpu.sync_copy(
            acc.at[pl.ds(sid * rpt, rpt)],
            out_hbm.at[cid].at[pl.ds(sid * rpt, rpt)],
        )

    return deg_kernel(dst)


def _sc_edge_pass(y, eidx):
    n_nodes, d = y.shape
    del n_nodes
    nwin = eidx.shape[1]
    k = eidx.shape[3]
    rpt = NPAD // NS
    nb = 5
    nph = 10
    nphase = nwin // nph

    @functools.partial(
        pl.kernel,
        mesh=_mesh(),
        out_type=jax.ShapeDtypeStruct((NC, NPAD, d), jnp.float32),
        scratch_types=[
            pltpu.VMEM_SHARED((NPAD, d), jnp.float32),
            pltpu.VMEM((2, nphase, 2, k), jnp.int32),
            pltpu.VMEM((nb, k, d), jnp.float32),
        ] + [pltpu.SemaphoreType.DMA] * (2 * nb + 2),
    )
    def edge_kernel(y_hbm, eidx_hbm, out_hbm, acc, idxs, rows, *sems):
        gsem, ssem, isem = sems[:nb], sems[nb:2 * nb], sems[2 * nb:]
        cid = lax.axis_index("c")
        sid = lax.axis_index("s")
        wid = cid * NS + sid

        def idx_start(pb, p):
            pltpu.async_copy(
                eidx_hbm.at[wid].at[pl.ds(p * nphase, nphase)],
                idxs.at[pb], isem[pb])

        def idx_wait(pb, p):
            pltpu.make_async_copy(
                eidx_hbm.at[wid].at[pl.ds(p * nphase, nphase)],
                idxs.at[pb], isem[pb]).wait()

        idx_start(0, 0)

        @pl.loop(0, k)
        def _(r):
            @pl.loop(0, d // LANES)
            def _(cc):
                rows[0, r, pl.ds(cc * LANES, LANES)] = jnp.zeros(
                    (LANES,), jnp.float32)

        @pl.loop(0, rpt // k)
        def _(z):
            pltpu.sync_copy(rows.at[0], acc.at[pl.ds(sid * rpt + z * k, k)])

        plsc.subcore_barrier()

        def gath_start(pb, b, w):
            pltpu.async_copy(y_hbm.at[idxs.at[pb].at[w].at[0]], rows.at[b],
                             gsem[b])

        def gath_wait(pb, b, w):
            pltpu.make_async_copy(y_hbm.at[idxs.at[pb].at[w].at[0]],
                                  rows.at[b], gsem[b]).wait()

        def scat_start(pb, b, w):
            pltpu.async_copy(rows.at[b], acc.at[idxs.at[pb].at[w].at[1]],
                             ssem[b], add=True)

        def scat_wait(pb, b, w):
            pltpu.make_async_copy(rows.at[b], acc.at[idxs.at[pb].at[w].at[1]],
                                  ssem[b]).wait()

        for p in range(nph):
            pb = p % 2
            idx_wait(pb, p)
            if p + 1 < nph:
                idx_start(1 - pb, p + 1)

            for b in range(nb):
                gath_start(pb, b, b)

            @pl.loop(0, nphase // nb - 1)
            def _(g):
                w0 = g * nb
                for b in range(nb):
                    gath_wait(pb, b, w0 + b)
                    scat_start(pb, b, w0 + b)
                for b in range(nb):
                    scat_wait(pb, b, w0 + b)
                    gath_start(pb, b, w0 + nb + b)

            wl = nphase - nb
            for b in range(nb):
                gath_wait(pb, b, wl + b)
                scat_start(pb, b, wl + b)
            for b in range(nb):
                scat_wait(pb, b, wl + b)

        plsc.subcore_barrier()
        pltpu.sync_copy(
            acc.at[pl.ds(sid * rpt, rpt)],
            out_hbm.at[cid].at[pl.ds(sid * rpt, rpt)],
        )

    return edge_kernel(y, eidx)


_R = 1000


def _tc_scale_matmul(x, w, degc):
    n, d = x.shape

    def body(x_ref, w_ref, d0_ref, d1_ref, y_ref, dinv_ref):
        deg = d0_ref[0, :, 0:1] + d1_ref[0, :, 0:1] + 2.0
        dinv = lax.rsqrt(deg)
        dinv_ref[...] = dinv
        xw = jnp.dot(x_ref[...], w_ref[...],
                     preferred_element_type=jnp.float32)
        y_ref[...] = xw * dinv

    return pl.pallas_call(
        body,
        grid=(n // _R,),
        in_specs=[
            pl.BlockSpec((_R, d), lambda i: (i, 0)),
            pl.BlockSpec((d, d), lambda i: (0, 0)),
            pl.BlockSpec((1, _R, LANES), lambda i: (0, i, 0)),
            pl.BlockSpec((1, _R, LANES), lambda i: (1, i, 0)),
        ],
        out_specs=[
            pl.BlockSpec((_R, d), lambda i: (i, 0)),
            pl.BlockSpec((_R, 1), lambda i: (i, 0)),
        ],
        out_shape=[
            jax.ShapeDtypeStruct((n, d), jnp.float32),
            jax.ShapeDtypeStruct((n, 1), jnp.float32),
        ],
    )(x, w, degc, degc)


def _tc_combine_matmul(part, y, res, w, b, dinv):
    n, d = y.shape

    def body(p0_ref, p1_ref, y_ref, res_ref, w_ref, b_ref, dinv_ref,
             h_ref, y2_ref):
        dinv = dinv_ref[...]
        h = dinv * (p0_ref[0] + p1_ref[0] + 2.0 * y_ref[...]) \
            + b_ref[...][None, :] + res_ref[...]
        h_ref[...] = h
        y2_ref[...] = jnp.dot(h, w_ref[...],
                              preferred_element_type=jnp.float32) * dinv

    return pl.pallas_call(
        body,
        grid=(n // _R,),
        in_specs=[
            pl.BlockSpec((1, _R, d), lambda i: (0, i, 0)),
            pl.BlockSpec((1, _R, d), lambda i: (1, i, 0)),
            pl.BlockSpec((_R, d), lambda i: (i, 0)),
            pl.BlockSpec((_R, d), lambda i: (i, 0)),
            pl.BlockSpec((d, d), lambda i: (0, 0)),
            pl.BlockSpec((d,), lambda i: (0,)),
            pl.BlockSpec((_R, 1), lambda i: (i, 0)),
        ],
        out_specs=[
            pl.BlockSpec((_R, d), lambda i: (i, 0)),
            pl.BlockSpec((_R, d), lambda i: (i, 0)),
        ],
        out_shape=[
            jax.ShapeDtypeStruct((n, d), jnp.float32),
            jax.ShapeDtypeStruct((n, d), jnp.float32),
        ],
    )(part, part, y, res, w, b, dinv)


def _tc_combine(part, y, res, b, dinv):
    n, d = y.shape

    def body(p0_ref, p1_ref, y_ref, res_ref, b_ref, dinv_ref, o_ref):
        o_ref[...] = dinv_ref[...] * (p0_ref[0] + p1_ref[0]
                                      + 2.0 * y_ref[...]) \
            + b_ref[...][None, :] + res_ref[...]

    return pl.pallas_call(
        body,
        grid=(n // _R,),
        in_specs=[
            pl.BlockSpec((1, _R, d), lambda i: (0, i, 0)),
            pl.BlockSpec((1, _R, d), lambda i: (1, i, 0)),
            pl.BlockSpec((_R, d), lambda i: (i, 0)),
            pl.BlockSpec((_R, d), lambda i: (i, 0)),
            pl.BlockSpec((d,), lambda i: (0,)),
            pl.BlockSpec((_R, 1), lambda i: (i, 0)),
        ],
        out_specs=pl.BlockSpec((_R, d), lambda i: (i, 0)),
        out_shape=jax.ShapeDtypeStruct((n, d), jnp.float32),
    )(part, part, y, res, b, dinv)


def kernel(x, edge_index, W1, b1, W2, b2):
    e = edge_index.shape[1]
    k = 40
    nwin = e // (NW * k)
    eidx = jnp.stack(
        [edge_index[0].reshape(NW, nwin, k),
         edge_index[1].reshape(NW, nwin, k)], axis=2)

    degc = _sc_degree(edge_index[1].reshape(NW, nwin, k), x.shape[0])
    y1, dinv = _tc_scale_matmul(x, W1, degc)
    p1 = _sc_edge_pass(y1, eidx)
    h, y2 = _tc_combine_matmul(p1, y1, x, W2, b1, dinv)
    p2 = _sc_edge_pass(y2, eidx)
    out = _tc_combine(p2, y2, h, b2, dinv)
    return out


# speedup vs baseline: 1.0324x; 1.0095x over previous
import functools

import jax
import jax.numpy as jnp
from jax import lax
from jax.experimental import pallas as pl
from jax.experimental.pallas import tpu as pltpu
from jax.experimental.pallas import tpu_sc as plsc

NC = 2
NS = 16
NW = NC * NS
LANES = 16
NPAD = 10240

_mesh = lambda: plsc.VectorSubcoreMesh(core_axis_name="c", subcore_axis_name="s")


def _sc_degree(dst, n_nodes):
    del n_nodes
    nwin = dst.shape[1]
    k = dst.shape[2]
    rpt = NPAD // NS
    nb = 4

    @functools.partial(
        pl.kernel,
        mesh=_mesh(),
        out_type=jax.ShapeDtypeStruct((NC, NPAD, LANES), jnp.float32),
        scratch_types=[
            pltpu.VMEM_SHARED((NPAD, LANES), jnp.float32),
            pltpu.VMEM((rpt, LANES), jnp.float32),
            pltpu.VMEM((k, LANES), jnp.float32),
            pltpu.VMEM((nwin, k), jnp.int32),
        ] + [pltpu.SemaphoreType.DMA] * nb,
    )
    def deg_kernel(dst_hbm, out_hbm, acc, zbuf, ones_v, idxall, *ssem):
        cid = lax.axis_index("c")
        sid = lax.axis_index("s")
        wid = cid * NS + sid

        @pl.loop(0, rpt)
        def _(r):
            zbuf[r, :] = jnp.zeros((LANES,), jnp.float32)

        @pl.loop(0, k)
        def _(r):
            ones_v[r, :] = jnp.ones((LANES,), jnp.float32)

        pltpu.sync_copy(dst_hbm.at[wid], idxall)
        pltpu.sync_copy(zbuf, acc.at[pl.ds(sid * rpt, rpt)])
        plsc.subcore_barrier()

        def scat_start(b, w):
            pltpu.async_copy(ones_v, acc.at[idxall.at[w]], ssem[b],
                             add=True)

        def scat_wait(b, w):
            pltpu.make_async_copy(ones_v, acc.at[idxall.at[w]],
                                  ssem[b]).wait()

        for b in range(nb):
            scat_start(b, b)

        @pl.loop(0, nwin // nb - 1)
        def _(g):
            w0 = g * nb
            for b in range(nb):
                scat_wait(b, w0 + b)
                scat_start(b, w0 + nb + b)

        wl = nwin - nb
        for b in range(nb):
            scat_wait(b, wl + b)

        plsc.subcore_barrier()
        pltpu.sync_copy(
            acc.at[pl.ds(sid * rpt, rpt)],
            out_hbm.at[cid].at[pl.ds(sid * rpt, rpt)],
        )

    return deg_kernel(dst)


def _sc_edge_pass(y, eidx):
    n_nodes, d = y.shape
    del n_nodes
    nwin = eidx.shape[1]
    k = eidx.shape[3]
    rpt = NPAD // NS
    nb = 4
    nph = 16
    nphase = nwin // nph

    @functools.partial(
        pl.kernel,
        mesh=_mesh(),
        out_type=jax.ShapeDtypeStruct((NC, NPAD, d), jnp.float32),
        scratch_types=[
            pltpu.VMEM_SHARED((NPAD, d), jnp.float32),
            pltpu.VMEM((2, nphase, 2, k), jnp.int32),
            pltpu.VMEM((nb, k, d), jnp.float32),
        ] + [pltpu.SemaphoreType.DMA] * (2 * nb + 2),
    )
    def edge_kernel(y_hbm, eidx_hbm, out_hbm, acc, idxs, rows, *sems):
        gsem, ssem, isem = sems[:nb], sems[nb:2 * nb], sems[2 * nb:]
        cid = lax.axis_index("c")
        sid = lax.axis_index("s")
        wid = cid * NS + sid

        def idx_start(pb, p):
            pltpu.async_copy(
                eidx_hbm.at[wid].at[pl.ds(p * nphase, nphase)],
                idxs.at[pb], isem[pb])

        def idx_wait(pb, p):
            pltpu.make_async_copy(
                eidx_hbm.at[wid].at[pl.ds(p * nphase, nphase)],
                idxs.at[pb], isem[pb]).wait()

        idx_start(0, 0)

        @pl.loop(0, k)
        def _(r):
            @pl.loop(0, d // LANES)
            def _(cc):
                rows[0, r, pl.ds(cc * LANES, LANES)] = jnp.zeros(
                    (LANES,), jnp.float32)

        @pl.loop(0, rpt // k)
        def _(z):
            pltpu.sync_copy(rows.at[0], acc.at[pl.ds(sid * rpt + z * k, k)])

        plsc.subcore_barrier()

        def gath_start(pb, b, w):
            pltpu.async_copy(y_hbm.at[idxs.at[pb].at[w].at[0]], rows.at[b],
                             gsem[b])

        def gath_wait(pb, b, w):
            pltpu.make_async_copy(y_hbm.at[idxs.at[pb].at[w].at[0]],
                                  rows.at[b], gsem[b]).wait()

        def scat_start(pb, b, w):
            pltpu.async_copy(rows.at[b], acc.at[idxs.at[pb].at[w].at[1]],
                             ssem[b], add=True)

        def scat_wait(pb, b, w):
            pltpu.make_async_copy(rows.at[b], acc.at[idxs.at[pb].at[w].at[1]],
                                  ssem[b]).wait()

        for p in range(nph):
            pb = p % 2
            idx_wait(pb, p)
            if p + 1 < nph:
                idx_start(1 - pb, p + 1)

            for b in range(nb):
                gath_start(pb, b, b)

            @pl.loop(0, nphase // nb - 1)
            def _(g):
                w0 = g * nb
                for b in range(nb):
                    gath_wait(pb, b, w0 + b)
                    scat_start(pb, b, w0 + b)
                for b in range(nb):
                    scat_wait(pb, b, w0 + b)
                    gath_start(pb, b, w0 + nb + b)

            wl = nphase - nb
            for b in range(nb):
                gath_wait(pb, b, wl + b)
                scat_start(pb, b, wl + b)
            for b in range(nb):
                scat_wait(pb, b, wl + b)

        plsc.subcore_barrier()
        pltpu.sync_copy(
            acc.at[pl.ds(sid * rpt, rpt)],
            out_hbm.at[cid].at[pl.ds(sid * rpt, rpt)],
        )

    return edge_kernel(y, eidx)


_R = 1000


def _tc_scale_matmul(x, w, degc):
    n, d = x.shape

    def body(x_ref, w_ref, d0_ref, d1_ref, y_ref, dinv_ref):
        deg = d0_ref[0, :, 0:1] + d1_ref[0, :, 0:1] + 2.0
        dinv = lax.rsqrt(deg)
        dinv_ref[...] = dinv
        xw = jnp.dot(x_ref[...], w_ref[...],
                     preferred_element_type=jnp.float32)
        y_ref[...] = xw * dinv

    return pl.pallas_call(
        body,
        grid=(n // _R,),
        in_specs=[
            pl.BlockSpec((_R, d), lambda i: (i, 0)),
            pl.BlockSpec((d, d), lambda i: (0, 0)),
            pl.BlockSpec((1, _R, LANES), lambda i: (0, i, 0)),
            pl.BlockSpec((1, _R, LANES), lambda i: (1, i, 0)),
        ],
        out_specs=[
            pl.BlockSpec((_R, d), lambda i: (i, 0)),
            pl.BlockSpec((_R, 1), lambda i: (i, 0)),
        ],
        out_shape=[
            jax.ShapeDtypeStruct((n, d), jnp.float32),
            jax.ShapeDtypeStruct((n, 1), jnp.float32),
        ],
    )(x, w, degc, degc)


def _tc_combine_matmul(part, y, res, w, b, dinv):
    n, d = y.shape

    def body(p0_ref, p1_ref, y_ref, res_ref, w_ref, b_ref, dinv_ref,
             h_ref, y2_ref):
        dinv = dinv_ref[...]
        h = dinv * (p0_ref[0] + p1_ref[0] + 2.0 * y_ref[...]) \
            + b_ref[...][None, :] + res_ref[...]
        h_ref[...] = h
        y2_ref[...] = jnp.dot(h, w_ref[...],
                              preferred_element_type=jnp.float32) * dinv

    return pl.pallas_call(
        body,
        grid=(n // _R,),
        in_specs=[
            pl.BlockSpec((1, _R, d), lambda i: (0, i, 0)),
            pl.BlockSpec((1, _R, d), lambda i: (1, i, 0)),
            pl.BlockSpec((_R, d), lambda i: (i, 0)),
            pl.BlockSpec((_R, d), lambda i: (i, 0)),
            pl.BlockSpec((d, d), lambda i: (0, 0)),
            pl.BlockSpec((d,), lambda i: (0,)),
            pl.BlockSpec((_R, 1), lambda i: (i, 0)),
        ],
        out_specs=[
            pl.BlockSpec((_R, d), lambda i: (i, 0)),
            pl.BlockSpec((_R, d), lambda i: (i, 0)),
        ],
        out_shape=[
            jax.ShapeDtypeStruct((n, d), jnp.float32),
            jax.ShapeDtypeStruct((n, d), jnp.float32),
        ],
    )(part, part, y, res, w, b, dinv)


def _tc_combine(part, y, res, b, dinv):
    n, d = y.shape

    def body(p0_ref, p1_ref, y_ref, res_ref, b_ref, dinv_ref, o_ref):
        o_ref[...] = dinv_ref[...] * (p0_ref[0] + p1_ref[0]
                                      + 2.0 * y_ref[...]) \
            + b_ref[...][None, :] + res_ref[...]

    return pl.pallas_call(
        body,
        grid=(n // _R,),
        in_specs=[
            pl.BlockSpec((1, _R, d), lambda i: (0, i, 0)),
            pl.BlockSpec((1, _R, d), lambda i: (1, i, 0)),
            pl.BlockSpec((_R, d), lambda i: (i, 0)),
            pl.BlockSpec((_R, d), lambda i: (i, 0)),
            pl.BlockSpec((d,), lambda i: (0,)),
            pl.BlockSpec((_R, 1), lambda i: (i, 0)),
        ],
        out_specs=pl.BlockSpec((_R, d), lambda i: (i, 0)),
        out_shape=jax.ShapeDtypeStruct((n, d), jnp.float32),
    )(part, part, y, res, b, dinv)


def kernel(x, edge_index, W1, b1, W2, b2):
    n = x.shape[0]
    e = edge_index.shape[1]
    k = 80
    ep = 10240
    epad = NW * ep - e
    src = jnp.concatenate(
        [edge_index[0],
         (jnp.arange(epad, dtype=jnp.int32) * 13) % jnp.int32(n)])
    dst = jnp.concatenate(
        [edge_index[1],
         jnp.int32(n) + (jnp.arange(epad, dtype=jnp.int32) % (NPAD - n))])
    nwin = ep // k
    eidx = jnp.stack(
        [src.reshape(NW, nwin, k), dst.reshape(NW, nwin, k)], axis=2)

    degc = _sc_degree(dst.reshape(NW, nwin, k), x.shape[0])
    y1, dinv = _tc_scale_matmul(x, W1, degc)
    p1 = _sc_edge_pass(y1, eidx)
    h, y2 = _tc_combine_matmul(p1, y1, x, W2, b1, dinv)
    p2 = _sc_edge_pass(y2, eidx)
    out = _tc_combine(p2, y2, h, b2, dinv)
    return out
